# Initial kernel scaffold; baseline (speedup 1.0000x reference)
import jax
import jax.numpy as jnp
from jax.experimental import pallas as pl


def kernel(x, gate_W, gate_b, fc_W, fc_b, out_W, out_b):
    raise NotImplementedError("write your pallas kernel here")


# speedup vs baseline: 3.4490x; 3.4490x over previous
import functools

import jax
import jax.numpy as jnp
from jax.experimental import pallas as pl
from jax.experimental.pallas import tpu as pltpu

HIDDEN = 256
FF = 640
E = 3
TILE = 256


def _moe_dense_kernel(x_ref, gw_ref, gb_ref, fcw_ref, fcb_ref,
                      ow_ref, ob_ref, out_ref, counts_ref, scores_ref):
    i = pl.program_id(0)

    @pl.when(i == 0)
    def _init():
        for e in range(E):
            counts_ref[e] = 0.0
            scores_ref[e] = 0.0

    x = x_ref[...]
    xb = x.astype(jnp.bfloat16)

    logits = jnp.dot(xb, gw_ref[...], preferred_element_type=jnp.float32)
    l0 = logits[:, 0] + gb_ref[0]
    l1 = logits[:, 1] + gb_ref[1]
    l2 = logits[:, 2] + gb_ref[2]
    m = jnp.maximum(jnp.maximum(l0, l1), l2)
    u0, u1, u2 = jnp.exp(l0 - m), jnp.exp(l1 - m), jnp.exp(l2 - m)
    s = u0 + u1 + u2
    p0, p1, p2 = u0 / s, u1 / s, u2 / s
    w = jnp.maximum(jnp.maximum(p0, p1), p2)
    eid = jnp.where(p0 >= p1,
                    jnp.where(p0 >= p2, 0, 2),
                    jnp.where(p1 >= p2, 1, 2)).astype(jnp.int32)

    acc = jnp.zeros((TILE, HIDDEN), dtype=jnp.float32)
    for e in range(E):
        h = jnp.dot(xb, fcw_ref[e], preferred_element_type=jnp.float32)
        h = h + fcb_ref[e, :].reshape(1, 2 * FF)
        x1 = h[:, :FF]
        x2 = h[:, FF:]
        g = x1 * (0.5 * x2 * (1.0 + jax.lax.erf(x2 * 0.7071067811865476)))
        y = jnp.dot(g.astype(jnp.bfloat16), ow_ref[e],
                    preferred_element_type=jnp.float32)
        y = y + ob_ref[e, :].reshape(1, HIDDEN)
        mask = (eid == e)
        acc = jnp.where(mask[:, None], y * w[:, None], acc)
        counts_ref[e] += jnp.sum(mask.astype(jnp.float32))
        scores_ref[e] += jnp.sum(jnp.where(mask, w, 0.0))
    out_ref[...] = acc


@jax.jit
def kernel(x, gate_W, gate_b, fc_W, fc_b, out_W, out_b):
    B, S, d = x.shape
    N = B * S
    x_flat = x.reshape(N, d)
    gate_Wt = gate_W.T.astype(jnp.bfloat16)
    fc_Wt = fc_W.transpose(0, 2, 1).astype(jnp.bfloat16)
    out_Wt = out_W.transpose(0, 2, 1).astype(jnp.bfloat16)

    grid = (N // TILE,)
    out, counts, scores = pl.pallas_call(
        _moe_dense_kernel,
        grid=grid,
        in_specs=[
            pl.BlockSpec((TILE, d), lambda i: (i, 0)),
            pl.BlockSpec((d, E), lambda i: (0, 0)),
            pl.BlockSpec(memory_space=pltpu.SMEM),
            pl.BlockSpec((E, d, 2 * FF), lambda i: (0, 0, 0)),
            pl.BlockSpec((E, 2 * FF), lambda i: (0, 0)),
            pl.BlockSpec((E, FF, d), lambda i: (0, 0, 0)),
            pl.BlockSpec((E, d), lambda i: (0, 0)),
        ],
        out_specs=[
            pl.BlockSpec((TILE, d), lambda i: (i, 0)),
            pl.BlockSpec(memory_space=pltpu.SMEM),
            pl.BlockSpec(memory_space=pltpu.SMEM),
        ],
        out_shape=[
            jax.ShapeDtypeStruct((N, d), jnp.float32),
            jax.ShapeDtypeStruct((E,), jnp.float32),
            jax.ShapeDtypeStruct((E,), jnp.float32),
        ],
    )(x_flat, gate_Wt, gate_b, fc_Wt, fc_b, out_Wt, out_b)

    usage = scores / (counts + 1e-08)
    loss = jnp.sum((usage - 1.0 / E) ** 2)
    return out.reshape(B, S, d), loss
